# speedup vs baseline: 1.3256x; 1.3256x over previous
import functools

import jax
import jax.numpy as jnp
from jax import lax
from jax.experimental import pallas as pl
from jax.experimental.pallas import tpu as pltpu
from jax.experimental.pallas import tpu_sc as plsc

BATCH = 4
SEQ = 8192
D_MODEL = 128
B = BATCH * SEQ
NC = 2
NS = 16
NW = NC * NS
B_PER_W = B // NW
CHUNK = 128
NCHUNK = B_PER_W // CHUNK


def _gather_kernel(idx_hbm, table_hbm, out_hbm, idx_v, rows_v, sem):
    wid = lax.axis_index("s") * NC + lax.axis_index("c")
    base = wid * B_PER_W
    pltpu.sync_copy(idx_hbm.at[pl.ds(base, B_PER_W)], idx_v)
    for i in range(NCHUNK):
        pltpu.async_copy(
            table_hbm.at[idx_v.at[pl.ds(i * CHUNK, CHUNK)]],
            rows_v,
            sem,
        ).wait()
        pltpu.sync_copy(rows_v, out_hbm.at[pl.ds(base + i * CHUNK, CHUNK)])


@jax.jit
def _embed(idx_flat, wte):
    mesh = plsc.VectorSubcoreMesh(core_axis_name="c", subcore_axis_name="s")
    k = functools.partial(
        pl.kernel,
        mesh=mesh,
        out_type=jax.ShapeDtypeStruct((B, D_MODEL), jnp.float32),
        scratch_types=[
            pltpu.VMEM((B_PER_W,), jnp.int32),
            pltpu.VMEM((CHUNK, D_MODEL), jnp.float32),
            pltpu.SemaphoreType.DMA,
        ],
    )(_gather_kernel)
    return k(idx_flat, wte)


def kernel(inputs, wte):
    out = _embed(inputs.reshape(B), wte)
    return out.reshape(BATCH, SEQ, D_MODEL)


# speedup vs baseline: 1.5148x; 1.1427x over previous
import functools

import jax
import jax.numpy as jnp
from jax import lax
from jax.experimental import pallas as pl
from jax.experimental.pallas import tpu as pltpu
from jax.experimental.pallas import tpu_sc as plsc

BATCH = 4
SEQ = 8192
D_MODEL = 128
B = BATCH * SEQ
NC = 2
NS = 16
NW = NC * NS
B_PER_W = B // NW
CHUNK = 128
NCHUNK = B_PER_W // CHUNK
NBUF = 4


def _gather_kernel(idx_hbm, table_hbm, out_hbm, idx_v, rows_v, gsems, wsems):
    wid = lax.axis_index("s") * NC + lax.axis_index("c")
    base = wid * B_PER_W
    pltpu.sync_copy(idx_hbm.at[pl.ds(base, B_PER_W)], idx_v)
    gd = [None] * NCHUNK
    wd = [None] * NCHUNK
    for i in range(NCHUNK):
        b = i % NBUF
        if i >= NBUF:
            wd[i - NBUF].wait()
        gd[i] = pltpu.async_copy(
            table_hbm.at[idx_v.at[pl.ds(i * CHUNK, CHUNK)]],
            rows_v.at[b],
            gsems.at[b],
        )
        if i >= 1:
            j = i - 1
            gd[j].wait()
            wd[j] = pltpu.async_copy(
                rows_v.at[j % NBUF],
                out_hbm.at[pl.ds(base + j * CHUNK, CHUNK)],
                wsems.at[j % NBUF],
            )
    last = NCHUNK - 1
    gd[last].wait()
    wd[last] = pltpu.async_copy(
        rows_v.at[last % NBUF],
        out_hbm.at[pl.ds(base + last * CHUNK, CHUNK)],
        wsems.at[last % NBUF],
    )
    for j in range(max(0, NCHUNK - NBUF), NCHUNK):
        wd[j].wait()


@jax.jit
def _embed(idx_flat, wte):
    mesh = plsc.VectorSubcoreMesh(core_axis_name="c", subcore_axis_name="s")
    k = functools.partial(
        pl.kernel,
        mesh=mesh,
        out_type=jax.ShapeDtypeStruct((B, D_MODEL), jnp.float32),
        scratch_types=[
            pltpu.VMEM((B_PER_W,), jnp.int32),
            pltpu.VMEM((NBUF, CHUNK, D_MODEL), jnp.float32),
            pltpu.SemaphoreType.DMA((NBUF,)),
            pltpu.SemaphoreType.DMA((NBUF,)),
        ],
    )(_gather_kernel)
    return k(idx_flat, wte)


def kernel(inputs, wte):
    out = _embed(inputs.reshape(B), wte)
    return out.reshape(BATCH, SEQ, D_MODEL)


# speedup vs baseline: 1.5273x; 1.0083x over previous
import functools

import jax
import jax.numpy as jnp
from jax import lax
from jax.experimental import pallas as pl
from jax.experimental.pallas import tpu as pltpu
from jax.experimental.pallas import tpu_sc as plsc

BATCH = 4
SEQ = 8192
D_MODEL = 128
B = BATCH * SEQ
NC = 2
NS = 16
NW = NC * NS
B_PER_W = B // NW
CHUNK = 128
NCHUNK = B_PER_W // CHUNK
NBUF = 6
LAG = 2


def _gather_kernel(idx_hbm, table_hbm, out_hbm, idx_v, rows_v, gsems, wsems):
    wid = lax.axis_index("s") * NC + lax.axis_index("c")
    base = wid * B_PER_W
    pltpu.sync_copy(idx_hbm.at[pl.ds(base, B_PER_W)], idx_v)
    gd = [None] * NCHUNK
    wd = [None] * NCHUNK

    def write_back(j):
        gd[j].wait()
        wd[j] = pltpu.async_copy(
            rows_v.at[j % NBUF],
            out_hbm.at[pl.ds(base + j * CHUNK, CHUNK)],
            wsems.at[j % NBUF],
        )

    for i in range(NCHUNK):
        b = i % NBUF
        if i >= NBUF:
            wd[i - NBUF].wait()
        gd[i] = pltpu.async_copy(
            table_hbm.at[idx_v.at[pl.ds(i * CHUNK, CHUNK)]],
            rows_v.at[b],
            gsems.at[b],
        )
        if i >= LAG:
            write_back(i - LAG)
    for j in range(max(0, NCHUNK - LAG), NCHUNK):
        write_back(j)
    for j in range(max(0, NCHUNK - NBUF), NCHUNK):
        wd[j].wait()


@jax.jit
def _embed(idx_flat, wte):
    mesh = plsc.VectorSubcoreMesh(core_axis_name="c", subcore_axis_name="s")
    k = functools.partial(
        pl.kernel,
        mesh=mesh,
        out_type=jax.ShapeDtypeStruct((B, D_MODEL), jnp.float32),
        scratch_types=[
            pltpu.VMEM((B_PER_W,), jnp.int32),
            pltpu.VMEM((NBUF, CHUNK, D_MODEL), jnp.float32),
            pltpu.SemaphoreType.DMA((NBUF,)),
            pltpu.SemaphoreType.DMA((NBUF,)),
        ],
    )(_gather_kernel)
    return k(idx_flat, wte)


def kernel(inputs, wte):
    out = _embed(inputs.reshape(B), wte)
    return out.reshape(BATCH, SEQ, D_MODEL)


# speedup vs baseline: 1.5293x; 1.0013x over previous
import functools

import jax
import jax.numpy as jnp
from jax import lax
from jax.experimental import pallas as pl
from jax.experimental.pallas import tpu as pltpu
from jax.experimental.pallas import tpu_sc as plsc

BATCH = 4
SEQ = 8192
D_MODEL = 128
B = BATCH * SEQ
NC = 2
NS = 16
NW = NC * NS
B_PER_W = B // NW
CHUNK = 128
NCHUNK = B_PER_W // CHUNK
NBUF = 6
LAG = 2


W_PER_ROW = SEQ // B_PER_W


def _gather_kernel(idx_hbm, table_hbm, out_hbm, idx_v, rows_v, gsems, wsems):
    wid = lax.axis_index("s") * NC + lax.axis_index("c")
    base = wid * B_PER_W
    row = wid // W_PER_ROW
    col = (wid % W_PER_ROW) * B_PER_W
    pltpu.sync_copy(idx_hbm.at[row, pl.ds(col, B_PER_W)], idx_v)
    gd = [None] * NCHUNK
    wd = [None] * NCHUNK

    def write_back(j):
        gd[j].wait()
        wd[j] = pltpu.async_copy(
            rows_v.at[j % NBUF],
            out_hbm.at[pl.ds(base + j * CHUNK, CHUNK)],
            wsems.at[j % NBUF],
        )

    for i in range(NCHUNK):
        b = i % NBUF
        if i >= NBUF:
            wd[i - NBUF].wait()
        gd[i] = pltpu.async_copy(
            table_hbm.at[idx_v.at[pl.ds(i * CHUNK, CHUNK)]],
            rows_v.at[b],
            gsems.at[b],
        )
        if i >= LAG:
            write_back(i - LAG)
    for j in range(max(0, NCHUNK - LAG), NCHUNK):
        write_back(j)
    for j in range(max(0, NCHUNK - NBUF), NCHUNK):
        wd[j].wait()


@jax.jit
def _embed(inputs, wte):
    mesh = plsc.VectorSubcoreMesh(core_axis_name="c", subcore_axis_name="s")
    k = functools.partial(
        pl.kernel,
        mesh=mesh,
        out_type=jax.ShapeDtypeStruct((B, D_MODEL), jnp.float32),
        scratch_types=[
            pltpu.VMEM((B_PER_W,), jnp.int32),
            pltpu.VMEM((NBUF, CHUNK, D_MODEL), jnp.float32),
            pltpu.SemaphoreType.DMA((NBUF,)),
            pltpu.SemaphoreType.DMA((NBUF,)),
        ],
    )(_gather_kernel)
    return k(inputs, wte)


def kernel(inputs, wte):
    out = _embed(inputs, wte)
    return out.reshape(BATCH, SEQ, D_MODEL)


# speedup vs baseline: 1.5560x; 1.0175x over previous
import functools

import jax
import jax.numpy as jnp
from jax import lax
from jax.experimental import pallas as pl
from jax.experimental.pallas import tpu as pltpu
from jax.experimental.pallas import tpu_sc as plsc

BATCH = 4
SEQ = 8192
D_MODEL = 128
B = BATCH * SEQ
NC = 2
NS = 16
NW = NC * NS
B_PER_W = B // NW
CHUNK = 256
NCHUNK = B_PER_W // CHUNK
NBUF = 3
LAG = 2
W_PER_ROW = SEQ // B_PER_W


def _gather_kernel(idx_hbm, table_hbm, out_hbm, idx_v, rows_v, gsems, wsems):
    wid = lax.axis_index("s") * NC + lax.axis_index("c")
    base = wid * B_PER_W
    row = wid // W_PER_ROW
    col = (wid % W_PER_ROW) * B_PER_W
    pltpu.sync_copy(idx_hbm.at[row, pl.ds(col, B_PER_W)], idx_v)
    gd = [None] * NCHUNK
    wd = [None] * NCHUNK

    def write_back(j):
        gd[j].wait()
        wd[j] = pltpu.async_copy(
            rows_v.at[j % NBUF],
            out_hbm.at[pl.ds(base + j * CHUNK, CHUNK)],
            wsems.at[j % NBUF],
        )

    for i in range(NCHUNK):
        b = i % NBUF
        if i >= NBUF:
            wd[i - NBUF].wait()
        gd[i] = pltpu.async_copy(
            table_hbm.at[idx_v.at[pl.ds(i * CHUNK, CHUNK)]],
            rows_v.at[b],
            gsems.at[b],
        )
        if i >= LAG:
            write_back(i - LAG)
    for j in range(max(0, NCHUNK - LAG), NCHUNK):
        write_back(j)
    for j in range(max(0, NCHUNK - NBUF), NCHUNK):
        wd[j].wait()


@jax.jit
def _embed(inputs, wte):
    mesh = plsc.VectorSubcoreMesh(core_axis_name="c", subcore_axis_name="s")
    k = functools.partial(
        pl.kernel,
        mesh=mesh,
        out_type=jax.ShapeDtypeStruct((B, D_MODEL), jnp.float32),
        scratch_types=[
            pltpu.VMEM((B_PER_W,), jnp.int32),
            pltpu.VMEM((NBUF, CHUNK, D_MODEL), jnp.float32),
            pltpu.SemaphoreType.DMA((NBUF,)),
            pltpu.SemaphoreType.DMA((NBUF,)),
        ],
    )(_gather_kernel)
    return k(inputs, wte)


def kernel(inputs, wte):
    out = _embed(inputs, wte)
    return out.reshape(BATCH, SEQ, D_MODEL)
